# baseline (speedup 1.0000x reference)
import dataclasses
import functools

import jax
import jax.numpy as jnp
from jax import lax
from jax.experimental import pallas as pl
from jax.experimental.pallas import tpu as pltpu
from jax.experimental.pallas import tpu_sc as plsc

N = 10000
E = 320000
D = 128
H = 128
OUT = 128

NCORES = 2
NSUB = 16
NTILES = NCORES * NSUB
EPT = E // NTILES
CH = 64
NCH0 = 156
XTRA = 2
NPAD = 10240
ROWS_PT = NPAD // NSUB
DR = NPAD // 128


def _proj_body(x_ref, ws_ref, wd_ref, atts_ref, attd_ref,
               h_ref, as_ref, ad_ref):
    xb = x_ref[...]
    h = jnp.dot(xb, ws_ref[...], preferred_element_type=jnp.float32)
    h_ref[...] = h
    as_ref[...] = jnp.dot(h, atts_ref[...], preferred_element_type=jnp.float32)
    hd = jnp.dot(xb, wd_ref[...], preferred_element_type=jnp.float32)
    ad_ref[...] = jnp.dot(hd, attd_ref[...], preferred_element_type=jnp.float32)


def _tc_proj(x, W_src, W_dst, att_src, att_dst):
    R = 1000
    grid = (N // R,)
    return pl.pallas_call(
        _proj_body,
        grid=grid,
        in_specs=[
            pl.BlockSpec((R, D), lambda i: (i, 0)),
            pl.BlockSpec((D, H), lambda i: (0, 0)),
            pl.BlockSpec((D, H), lambda i: (0, 0)),
            pl.BlockSpec((H, 1), lambda i: (0, 0)),
            pl.BlockSpec((H, 1), lambda i: (0, 0)),
        ],
        out_specs=[
            pl.BlockSpec((R, H), lambda i: (i, 0)),
            pl.BlockSpec((R, 1), lambda i: (i, 0)),
            pl.BlockSpec((R, 1), lambda i: (i, 0)),
        ],
        out_shape=[
            jax.ShapeDtypeStruct((N, H), jnp.float32),
            jax.ShapeDtypeStruct((N, 1), jnp.float32),
            jax.ShapeDtypeStruct((N, 1), jnp.float32),
        ],
    )(x, W_src, W_dst, att_src, att_dst)


def _final_body(acc_ref, den_ref, bias_ref, wl_ref, bl_ref, out_ref):
    num = acc_ref[0] + acc_ref[1]
    den = den_ref[0] + den_ref[1]
    node = jnp.maximum(num / (den + 1e-16) + bias_ref[...], 0.0)
    out_ref[...] = (
        jnp.dot(node, wl_ref[...], preferred_element_type=jnp.float32)
        + bl_ref[...]
    )


def _tc_final(acc, den, bias_conv, W_lin, b_lin):
    R = 1000
    grid = (N // R,)
    return pl.pallas_call(
        _final_body,
        grid=grid,
        in_specs=[
            pl.BlockSpec((NCORES, R, H), lambda i: (0, i, 0)),
            pl.BlockSpec((NCORES, R, 1), lambda i: (0, i, 0)),
            pl.BlockSpec((1, H), lambda i: (0, 0)),
            pl.BlockSpec((H, OUT), lambda i: (0, 0)),
            pl.BlockSpec((1, OUT), lambda i: (0, 0)),
        ],
        out_specs=pl.BlockSpec((R, OUT), lambda i: (i, 0)),
        out_shape=jax.ShapeDtypeStruct((N, OUT), jnp.float32),
    )(acc, den, bias_conv, W_lin, b_lin)


def _sc_edges(h_src, a_src, a_dst, src, dst):
    mesh = plsc.VectorSubcoreMesh(core_axis_name="c", subcore_axis_name="s")
    cp = pltpu.CompilerParams()
    if "needs_layout_passes" in pltpu.CompilerParams.__dataclass_fields__:
        cp = dataclasses.replace(cp, needs_layout_passes=False)

    @functools.partial(
        pl.kernel,
        out_type=[jax.ShapeDtypeStruct((NCORES, NPAD, H), jnp.float32),
                  jax.ShapeDtypeStruct((NCORES, DR, H), jnp.float32)],
        mesh=mesh,
        compiler_params=cp,
        scratch_types=[
            pltpu.VMEM((2 * CH,), jnp.int32),
            pltpu.VMEM((2 * CH,), jnp.int32),
            pltpu.VMEM((CH,), jnp.int32),
            pltpu.VMEM((CH,), jnp.int32),
            pltpu.VMEM((CH,), jnp.int32),
            pltpu.VMEM((CH,), jnp.int32),
            pltpu.VMEM((N,), jnp.float32),
            pltpu.VMEM((N,), jnp.float32),
            pltpu.VMEM((2 * CH,), jnp.float32),
            pltpu.VMEM((CH, H), jnp.float32),
            pltpu.VMEM((CH, H), jnp.float32),
            pltpu.VMEM((DR, H), jnp.float32),
            pltpu.VMEM((DR,), jnp.int32),
            pltpu.SemaphoreType.DMA,
            pltpu.SemaphoreType.DMA,
            pltpu.SemaphoreType.DMA,
            pltpu.SemaphoreType.DMA,
            pltpu.SemaphoreType.DMA,
            pltpu.SemaphoreType.DMA,
            pltpu.VMEM_SHARED((NPAD, H), jnp.float32),
            pltpu.VMEM_SHARED((DR, H), jnp.float32),
        ],
    )
    def edge_kernel(h_hbm, asrc_hbm, adst_hbm, src_hbm, dst_hbm,
                    out_hbm, dout_hbm,
                    srcA_v, srcB_v, dstA0_v, dstA1_v, dstB0_v, dstB1_v,
                    asrc_v, adst_v,
                    ps_v, rows0_v, rows1_v, den_v, ident_v,
                    sem_g0, sem_g1, sem_s0, sem_s1, sem_iA, sem_iB,
                    acc_sh, den_sh):
        cid = lax.axis_index("c")
        sid = lax.axis_index("s")

        pltpu.sync_copy(asrc_hbm, asrc_v)
        pltpu.sync_copy(adst_hbm, adst_v)

        @pl.loop(0, CH)
        def _(j):
            for c in range(0, H, 16):
                rows0_v[j, pl.ds(c, 16)] = jnp.zeros((16,), jnp.float32)

        @pl.loop(0, DR)
        def _(j):
            for c in range(0, H, 16):
                den_v[j, pl.ds(c, 16)] = jnp.zeros((16,), jnp.float32)

        lane16 = lax.iota(jnp.int32, 16)

        @pl.loop(0, DR, step=16)
        def _(k):
            ident_v[pl.ds(k, 16)] = lane16 + k

        row0 = sid * ROWS_PT

        @pl.loop(0, ROWS_PT, step=CH)
        def _(r):
            pltpu.sync_copy(rows0_v, acc_sh.at[pl.ds(row0 + r, CH)])

        @pl.when(sid < DR // 8)
        def _():
            pltpu.sync_copy(rows0_v.at[pl.ds(0, 8)],
                            den_sh.at[pl.ds(sid * 8, 8)])

        def _vmaxs(i, m):
            return jnp.maximum(m, asrc_v[pl.ds(i * 16, 16)])
        msrc = lax.fori_loop(0, N // 16, _vmaxs,
                             jnp.full((16,), -1e30, jnp.float32))

        def _vmaxd(i, m):
            return jnp.maximum(m, adst_v[pl.ds(i * 16, 16)])
        mdst = lax.fori_loop(0, N // 16, _vmaxd,
                             jnp.full((16,), -1e30, jnp.float32))
        g = jnp.maximum(
            lax.reduce_max(msrc, (0,)) + lax.reduce_max(mdst, (0,)), 0.0)
        gvec = jnp.zeros((16,), jnp.float32) + g

        plsc.subcore_barrier()

        tid = cid * NSUB + sid
        nch = NCH0 + 4 * jnp.where(tid < XTRA, 1, 0)
        tile_base = tid * (NCH0 * CH) + 4 * CH * jnp.minimum(tid, XTRA)
        sem_g = (sem_g0, sem_g1)
        sem_s = (sem_s0, sem_s1)
        rows_b = (rows0_v, rows1_v)
        src_p = (srcA_v, srcB_v)
        dst_p = ((dstA0_v, dstA1_v), (dstB0_v, dstB1_v))
        sem_i = (sem_iA, sem_iB)

        def _idx_descs(chunk, pp):
            base = jnp.minimum(tile_base + chunk * CH, E - 2 * CH)
            return (
                pltpu.make_async_copy(
                    src_hbm.at[pl.ds(base, 2 * CH)], src_p[pp], sem_i[pp]),
                pltpu.make_async_copy(
                    dst_hbm.at[pl.ds(base, CH)], dst_p[pp][0], sem_i[pp]),
                pltpu.make_async_copy(
                    dst_hbm.at[pl.ds(base + CH, CH)], dst_p[pp][1],
                    sem_i[pp]),
            )

        def _idx_start(chunk, pp):
            for d in _idx_descs(chunk, pp):
                d.start()

        def _idx_wait(chunk, pp):
            for d in _idx_descs(chunk, pp):
                d.wait()

        def _gather_start(b, pp):
            pltpu.async_copy(h_hbm.at[src_p[pp].at[pl.ds(b * CH, CH)]],
                             rows_b[b], sem_g[b])

        def _gather_wait(b, pp):
            pltpu.make_async_copy(h_hbm.at[src_p[pp].at[pl.ds(b * CH, CH)]],
                                  rows_b[b], sem_g[b]).wait()

        def _scatter_start(b, pp):
            pltpu.async_copy(rows_b[b], acc_sh.at[dst_p[pp][b]], sem_s[b],
                             add=True)

        def _scatter_wait(b, pp):
            pltpu.make_async_copy(rows_b[b], acc_sh.at[dst_p[pp][b]],
                                  sem_s[b]).wait()

        def _compute(b, pp):
            po = b * CH
            dref = dst_p[pp][b]

            @pl.loop(0, CH, step=16)
            def _(i):
                s16 = src_p[pp][pl.ds(po + i, 16)]
                d16 = dref[pl.ds(i, 16)]
                a_s = plsc.load_gather(asrc_v, [s16])
                a_d = plsc.load_gather(adst_v, [d16])
                e = a_s + a_d
                e = jnp.where(e > 0.0, e, e * 0.2)
                p = jnp.exp(e - gvec)
                ps_v[pl.ds(po + i, 16)] = p
                plsc.addupdate_scatter(
                    den_v, [lax.shift_right_logical(d16, 7),
                            lax.bitwise_and(d16, 127)], p)

            @plsc.parallel_loop(0, CH, unroll=8)
            def _(j):
                pj = plsc.load_gather(
                    ps_v, [jnp.zeros((16,), jnp.int32) + (po + j)])
                for c in range(0, H, 16):
                    rows_b[b][j, pl.ds(c, 16)] = (
                        rows_b[b][j, pl.ds(c, 16)] * pj)

        _idx_start(0, 0)
        _idx_wait(0, 0)
        _gather_start(0, 0)
        _gather_start(1, 0)
        _idx_start(2, 1)

        @pl.loop(0, nch, step=4)
        def _(k):
            _gather_wait(0, 0)
            _compute(0, 0)
            _scatter_start(0, 0)

            _gather_wait(1, 0)
            _compute(1, 0)
            _scatter_start(1, 0)

            _scatter_wait(0, 0)
            _idx_wait(k + 2, 1)
            _gather_start(0, 1)
            _scatter_wait(1, 0)
            _gather_start(1, 1)
            _idx_start(k + 4, 0)

            _gather_wait(0, 1)
            _compute(0, 1)
            _scatter_start(0, 1)

            _gather_wait(1, 1)
            _compute(1, 1)
            _scatter_start(1, 1)

            _scatter_wait(0, 1)
            _idx_wait(k + 4, 0)
            _gather_start(0, 0)
            _scatter_wait(1, 1)
            _gather_start(1, 0)
            _idx_start(k + 6, 1)

        _gather_wait(0, 0)
        _gather_wait(1, 0)
        _idx_wait(nch + 2, 1)

        pltpu.sync_copy(den_v, den_sh.at[ident_v], add=True)

        plsc.subcore_barrier()

        @pl.loop(0, ROWS_PT, step=2 * CH)
        def _(r):
            pltpu.sync_copy(acc_sh.at[pl.ds(row0 + r, 2 * CH)],
                            out_hbm.at[cid].at[pl.ds(row0 + r, 2 * CH)])

        @pl.when(sid < DR // 8)
        def _():
            pltpu.sync_copy(den_sh.at[pl.ds(sid * 8, 8)],
                            dout_hbm.at[cid].at[pl.ds(sid * 8, 8)])

    return edge_kernel(h_src, a_src, a_dst, src, dst)


def kernel(x, edge_index, W_src, W_dst, att_src, att_dst, bias_conv,
           W_lin, b_lin):
    ei = edge_index.astype(jnp.int32)
    src = ei[0]
    dst = ei[1]
    h_src, a_src, a_dst = _tc_proj(
        x, W_src, W_dst,
        att_src.reshape(H, 1).astype(jnp.float32),
        att_dst.reshape(H, 1).astype(jnp.float32))
    acc, dacc = _sc_edges(h_src, a_src.reshape(N), a_dst.reshape(N), src, dst)
    den = dacc.reshape(NCORES, NPAD)[:, :N].reshape(NCORES, N, 1)
    return _tc_final(acc, den, bias_conv.reshape(1, H), W_lin,
                     b_lin.reshape(1, OUT))


# speedup vs baseline: 1.0036x; 1.0036x over previous
import dataclasses
import functools

import jax
import jax.numpy as jnp
from jax import lax
from jax.experimental import pallas as pl
from jax.experimental.pallas import tpu as pltpu
from jax.experimental.pallas import tpu_sc as plsc

N = 10000
E = 320000
D = 128
H = 128
OUT = 128

NCORES = 2
NSUB = 16
NTILES = NCORES * NSUB
EPT = E // NTILES
CH = 64
NCH0 = 156
XTRA = 2
NPAD = 10240
ROWS_PT = NPAD // NSUB
DR = NPAD // 128


def _proj_body(x_ref, ws_ref, wd_ref, atts_ref, attd_ref,
               h_ref, as_ref, ad_ref):
    xb = x_ref[...]
    h = jnp.dot(xb, ws_ref[...], preferred_element_type=jnp.float32)
    h_ref[...] = h
    as_ref[...] = jnp.dot(h, atts_ref[...], preferred_element_type=jnp.float32)
    hd = jnp.dot(xb, wd_ref[...], preferred_element_type=jnp.float32)
    ad_ref[...] = jnp.dot(hd, attd_ref[...], preferred_element_type=jnp.float32)


def _tc_proj(x, W_src, W_dst, att_src, att_dst):
    R = 1000
    grid = (N // R,)
    return pl.pallas_call(
        _proj_body,
        grid=grid,
        in_specs=[
            pl.BlockSpec((R, D), lambda i: (i, 0)),
            pl.BlockSpec((D, H), lambda i: (0, 0)),
            pl.BlockSpec((D, H), lambda i: (0, 0)),
            pl.BlockSpec((H, 1), lambda i: (0, 0)),
            pl.BlockSpec((H, 1), lambda i: (0, 0)),
        ],
        out_specs=[
            pl.BlockSpec((R, H), lambda i: (i, 0)),
            pl.BlockSpec((R, 1), lambda i: (i, 0)),
            pl.BlockSpec((R, 1), lambda i: (i, 0)),
        ],
        out_shape=[
            jax.ShapeDtypeStruct((N, H), jnp.float32),
            jax.ShapeDtypeStruct((N, 1), jnp.float32),
            jax.ShapeDtypeStruct((N, 1), jnp.float32),
        ],
    )(x, W_src, W_dst, att_src, att_dst)


def _final_body(acc_ref, den_ref, bias_ref, wl_ref, bl_ref, out_ref):
    num = acc_ref[0] + acc_ref[1]
    den = den_ref[0] + den_ref[1]
    node = jnp.maximum(num / (den + 1e-16) + bias_ref[...], 0.0)
    out_ref[...] = (
        jnp.dot(node, wl_ref[...], preferred_element_type=jnp.float32)
        + bl_ref[...]
    )


def _tc_final(acc, den, bias_conv, W_lin, b_lin):
    R = 1000
    grid = (N // R,)
    return pl.pallas_call(
        _final_body,
        grid=grid,
        in_specs=[
            pl.BlockSpec((NCORES, R, H), lambda i: (0, i, 0)),
            pl.BlockSpec((NCORES, R, 1), lambda i: (0, i, 0)),
            pl.BlockSpec((1, H), lambda i: (0, 0)),
            pl.BlockSpec((H, OUT), lambda i: (0, 0)),
            pl.BlockSpec((1, OUT), lambda i: (0, 0)),
        ],
        out_specs=pl.BlockSpec((R, OUT), lambda i: (i, 0)),
        out_shape=jax.ShapeDtypeStruct((N, OUT), jnp.float32),
    )(acc, den, bias_conv, W_lin, b_lin)


def _sc_edges(h_src, a_src, a_dst, src, dst):
    mesh = plsc.VectorSubcoreMesh(core_axis_name="c", subcore_axis_name="s")
    cp = pltpu.CompilerParams()
    if "needs_layout_passes" in pltpu.CompilerParams.__dataclass_fields__:
        cp = dataclasses.replace(cp, needs_layout_passes=False)

    @functools.partial(
        pl.kernel,
        out_type=[jax.ShapeDtypeStruct((NCORES, NPAD, H), jnp.float32),
                  jax.ShapeDtypeStruct((NCORES, DR, H), jnp.float32)],
        mesh=mesh,
        compiler_params=cp,
        scratch_types=[
            pltpu.VMEM((2 * CH,), jnp.int32),
            pltpu.VMEM((2 * CH,), jnp.int32),
            pltpu.VMEM((CH,), jnp.int32),
            pltpu.VMEM((CH,), jnp.int32),
            pltpu.VMEM((CH,), jnp.int32),
            pltpu.VMEM((CH,), jnp.int32),
            pltpu.VMEM((N,), jnp.float32),
            pltpu.VMEM((N,), jnp.float32),
            pltpu.VMEM((2 * CH,), jnp.float32),
            pltpu.VMEM((CH, H), jnp.float32),
            pltpu.VMEM((CH, H), jnp.float32),
            pltpu.VMEM((DR, H), jnp.float32),
            pltpu.VMEM((DR,), jnp.int32),
            pltpu.SemaphoreType.DMA,
            pltpu.SemaphoreType.DMA,
            pltpu.SemaphoreType.DMA,
            pltpu.SemaphoreType.DMA,
            pltpu.SemaphoreType.DMA,
            pltpu.SemaphoreType.DMA,
            pltpu.VMEM_SHARED((NPAD, H), jnp.float32),
            pltpu.VMEM_SHARED((DR, H), jnp.float32),
        ],
    )
    def edge_kernel(h_hbm, asrc_hbm, adst_hbm, src_hbm, dst_hbm,
                    out_hbm, dout_hbm,
                    srcA_v, srcB_v, dstA0_v, dstA1_v, dstB0_v, dstB1_v,
                    asrc_v, adst_v,
                    ps_v, rows0_v, rows1_v, den_v, ident_v,
                    sem_g0, sem_g1, sem_s0, sem_s1, sem_iA, sem_iB,
                    acc_sh, den_sh):
        cid = lax.axis_index("c")
        sid = lax.axis_index("s")

        pltpu.sync_copy(asrc_hbm, asrc_v)
        pltpu.sync_copy(adst_hbm, adst_v)

        @pl.loop(0, CH)
        def _(j):
            for c in range(0, H, 16):
                rows0_v[j, pl.ds(c, 16)] = jnp.zeros((16,), jnp.float32)

        @pl.loop(0, DR)
        def _(j):
            for c in range(0, H, 16):
                den_v[j, pl.ds(c, 16)] = jnp.zeros((16,), jnp.float32)

        lane16 = lax.iota(jnp.int32, 16)

        @pl.loop(0, DR, step=16)
        def _(k):
            ident_v[pl.ds(k, 16)] = lane16 + k

        row0 = sid * ROWS_PT

        @pl.loop(0, ROWS_PT, step=CH)
        def _(r):
            pltpu.sync_copy(rows0_v, acc_sh.at[pl.ds(row0 + r, CH)])

        @pl.when(sid < DR // 8)
        def _():
            pltpu.sync_copy(rows0_v.at[pl.ds(0, 8)],
                            den_sh.at[pl.ds(sid * 8, 8)])

        def _vmaxs(i, m):
            return jnp.maximum(m, asrc_v[pl.ds(i * 16, 16)])
        msrc = lax.fori_loop(0, N // 16, _vmaxs,
                             jnp.full((16,), -1e30, jnp.float32))

        def _vmaxd(i, m):
            return jnp.maximum(m, adst_v[pl.ds(i * 16, 16)])
        mdst = lax.fori_loop(0, N // 16, _vmaxd,
                             jnp.full((16,), -1e30, jnp.float32))
        g = jnp.maximum(
            lax.reduce_max(msrc, (0,)) + lax.reduce_max(mdst, (0,)), 0.0)
        gvec = jnp.zeros((16,), jnp.float32) + g

        plsc.subcore_barrier()

        tid = cid * NSUB + sid
        nch = NCH0 + 4 * jnp.where(tid < XTRA, 1, 0)
        tile_base = tid * (NCH0 * CH) + 4 * CH * jnp.minimum(tid, XTRA)
        sem_g = (sem_g0, sem_g1)
        sem_s = (sem_s0, sem_s1)
        rows_b = (rows0_v, rows1_v)
        src_p = (srcA_v, srcB_v)
        dst_p = ((dstA0_v, dstA1_v), (dstB0_v, dstB1_v))
        sem_i = (sem_iA, sem_iB)

        def _idx_descs(chunk, pp):
            base = jnp.minimum(tile_base + chunk * CH, E - 2 * CH)
            return (
                pltpu.make_async_copy(
                    src_hbm.at[pl.ds(base, 2 * CH)], src_p[pp], sem_i[pp]),
                pltpu.make_async_copy(
                    dst_hbm.at[pl.ds(base, CH)], dst_p[pp][0], sem_i[pp]),
                pltpu.make_async_copy(
                    dst_hbm.at[pl.ds(base + CH, CH)], dst_p[pp][1],
                    sem_i[pp]),
            )

        def _idx_start(chunk, pp):
            for d in _idx_descs(chunk, pp):
                d.start()

        def _idx_wait(chunk, pp):
            for d in _idx_descs(chunk, pp):
                d.wait()

        def _gather_start(b, pp):
            pltpu.async_copy(h_hbm.at[src_p[pp].at[pl.ds(b * CH, CH)]],
                             rows_b[b], sem_g[b])

        def _gather_wait(b, pp):
            pltpu.make_async_copy(h_hbm.at[src_p[pp].at[pl.ds(b * CH, CH)]],
                                  rows_b[b], sem_g[b]).wait()

        def _scatter_start(b, pp):
            pltpu.async_copy(rows_b[b], acc_sh.at[dst_p[pp][b]], sem_s[b],
                             add=True)

        def _scatter_wait(b, pp):
            pltpu.make_async_copy(rows_b[b], acc_sh.at[dst_p[pp][b]],
                                  sem_s[b]).wait()

        def _compute(b, pp):
            po = b * CH
            dref = dst_p[pp][b]

            @pl.loop(0, CH, step=16)
            def _(i):
                s16 = src_p[pp][pl.ds(po + i, 16)]
                d16 = dref[pl.ds(i, 16)]
                a_s = plsc.load_gather(asrc_v, [s16])
                a_d = plsc.load_gather(adst_v, [d16])
                e = a_s + a_d
                e = jnp.where(e > 0.0, e, e * 0.2)
                p = jnp.exp(e - gvec)
                ps_v[pl.ds(po + i, 16)] = p
                plsc.addupdate_scatter(
                    den_v, [lax.shift_right_logical(d16, 7),
                            lax.bitwise_and(d16, 127)], p)

            @plsc.parallel_loop(0, CH, unroll=4)
            def _(j):
                pj = plsc.load_gather(
                    ps_v, [jnp.zeros((16,), jnp.int32) + (po + j)])
                for c in range(0, H, 16):
                    rows_b[b][j, pl.ds(c, 16)] = (
                        rows_b[b][j, pl.ds(c, 16)] * pj)

        _idx_start(0, 0)
        _idx_wait(0, 0)
        _gather_start(0, 0)
        _gather_start(1, 0)
        _idx_start(2, 1)

        @pl.loop(0, nch, step=4)
        def _(k):
            _gather_wait(0, 0)
            _compute(0, 0)
            _scatter_start(0, 0)

            _gather_wait(1, 0)
            _compute(1, 0)
            _scatter_start(1, 0)

            _scatter_wait(0, 0)
            _idx_wait(k + 2, 1)
            _gather_start(0, 1)
            _scatter_wait(1, 0)
            _gather_start(1, 1)
            _idx_start(k + 4, 0)

            _gather_wait(0, 1)
            _compute(0, 1)
            _scatter_start(0, 1)

            _gather_wait(1, 1)
            _compute(1, 1)
            _scatter_start(1, 1)

            _scatter_wait(0, 1)
            _idx_wait(k + 4, 0)
            _gather_start(0, 0)
            _scatter_wait(1, 1)
            _gather_start(1, 0)
            _idx_start(k + 6, 1)

        _gather_wait(0, 0)
        _gather_wait(1, 0)
        _idx_wait(nch + 2, 1)

        pltpu.sync_copy(den_v, den_sh.at[ident_v], add=True)

        plsc.subcore_barrier()

        @pl.loop(0, ROWS_PT, step=2 * CH)
        def _(r):
            pltpu.sync_copy(acc_sh.at[pl.ds(row0 + r, 2 * CH)],
                            out_hbm.at[cid].at[pl.ds(row0 + r, 2 * CH)])

        @pl.when(sid < DR // 8)
        def _():
            pltpu.sync_copy(den_sh.at[pl.ds(sid * 8, 8)],
                            dout_hbm.at[cid].at[pl.ds(sid * 8, 8)])

    return edge_kernel(h_src, a_src, a_dst, src, dst)


def kernel(x, edge_index, W_src, W_dst, att_src, att_dst, bias_conv,
           W_lin, b_lin):
    ei = edge_index.astype(jnp.int32)
    src = ei[0]
    dst = ei[1]
    h_src, a_src, a_dst = _tc_proj(
        x, W_src, W_dst,
        att_src.reshape(H, 1).astype(jnp.float32),
        att_dst.reshape(H, 1).astype(jnp.float32))
    acc, dacc = _sc_edges(h_src, a_src.reshape(N), a_dst.reshape(N), src, dst)
    den = dacc.reshape(NCORES, NPAD)[:, :N].reshape(NCORES, N, 1)
    return _tc_final(acc, den, bias_conv.reshape(1, H), W_lin,
                     b_lin.reshape(1, OUT))
